# baseline (speedup 1.0000x reference)
import functools
import jax
import jax.numpy as jnp
from jax.experimental import pallas as pl


def _edge_elemwise_body(ce_ref, dh_ref, eh_ref, bh_ref, ein_ref,
                        sigma_ref, msg_ref, eout_ref):
    e_new = ce_ref[...] + dh_ref[...] + eh_ref[...]
    sigma = jax.nn.sigmoid(e_new)
    sigma_ref[...] = sigma
    msg_ref[...] = sigma * bh_ref[...]
    eout_ref[...] = ein_ref[...] + jnp.maximum(e_new, 0.0)


def _edge_elemwise(ce, dh_g, eh_g, bh_g, e_in):
    E, F = ce.shape
    BE = 2000
    grid = (E // BE,)
    bs = pl.BlockSpec((BE, F), lambda i: (i, 0))
    out = [jax.ShapeDtypeStruct((E, F), jnp.float32)] * 3
    return pl.pallas_call(
        _edge_elemwise_body,
        grid=grid,
        in_specs=[bs] * 5,
        out_specs=[bs] * 3,
        out_shape=out,
    )(ce, dh_g, eh_g, bh_g, e_in)


def _linear(x, Wb):
    W, b = Wb
    return x @ W + b


def kernel(h, edge_index, edge_weight, params):
    N = h.shape[0]
    src = edge_index[0]
    dst = edge_index[1]
    e = edge_weight.reshape(-1, 1)
    h = _linear(h, params['emb_h'])
    e = _linear(e, params['emb_e'])
    for p in params['layers']:
        Ah = _linear(h, p['A'])
        Bh = _linear(h, p['B'])
        Dh = _linear(h, p['D'])
        Eh = _linear(h, p['E'])
        Ce = _linear(e, p['C'])
        sigma, msg, e_out = _edge_elemwise(
            Ce, jnp.take(Dh, src, axis=0), jnp.take(Eh, dst, axis=0),
            jnp.take(Bh, src, axis=0), e)
        num = jax.ops.segment_sum(msg, dst, num_segments=N)
        den = jax.ops.segment_sum(sigma, dst, num_segments=N)
        h = h + jax.nn.relu(Ah + num / (den + 1e-6))
        e = e_out
    return _linear(h, params['mlp'])


# speedup vs baseline: 1.1710x; 1.1710x over previous
import functools
import jax
import jax.numpy as jnp
from jax import lax
from jax.experimental import pallas as pl
from jax.experimental.pallas import tpu as pltpu
from jax.experimental.pallas import tpu_sc as plsc

NC = 2
NS = 16
F = 128
FH = 64
CHUNK = 80


def _node_body(emb, h_ref, wemb_ref, bemb_ref, wa_ref, ba_ref, wb_ref, bb_ref,
               wd_ref, bd_ref, we_ref, be_ref, h1_ref, ah_ref, db_ref, ehp_ref):
    x = h_ref[...]
    if emb:
        x = jnp.dot(x, wemb_ref[...], preferred_element_type=jnp.float32) \
            + bemb_ref[...]
        h1_ref[...] = x
    ah_ref[...] = jnp.dot(x, wa_ref[...], preferred_element_type=jnp.float32) \
        + ba_ref[...]
    bh = jnp.dot(x, wb_ref[...], preferred_element_type=jnp.float32) + bb_ref[...]
    dh = jnp.dot(x, wd_ref[...], preferred_element_type=jnp.float32) + bd_ref[...]
    eh = jnp.dot(x, we_ref[...], preferred_element_type=jnp.float32) + be_ref[...]
    db_ref[0] = jnp.concatenate([dh[:, :FH], bh[:, :FH]], axis=1)
    db_ref[1] = jnp.concatenate([dh[:, FH:], bh[:, FH:]], axis=1)
    ehp_ref[0] = eh[:, :FH]
    ehp_ref[1] = eh[:, FH:]


def _tc_node(h, p, emb_wb=None):
    N = h.shape[0]
    BN = 1000
    grid = (N // BN,)
    b2 = lambda r: pl.BlockSpec((BN, F), lambda i: (i, 0))
    w_spec = pl.BlockSpec((F, F), lambda i: (0, 0))
    bias_spec = pl.BlockSpec((1, F), lambda i: (0, 0))
    emb = emb_wb is not None
    if emb:
        wemb, bemb = emb_wb
    else:
        wemb = jnp.zeros((F, F), jnp.float32)
        bemb = jnp.zeros((F,), jnp.float32)
    outs = [
        jax.ShapeDtypeStruct((N, F), jnp.float32),
        jax.ShapeDtypeStruct((N, F), jnp.float32),
        jax.ShapeDtypeStruct((2, N, F), jnp.float32),
        jax.ShapeDtypeStruct((2, N, FH), jnp.float32),
    ]
    out_specs = [
        b2(0),
        b2(0),
        pl.BlockSpec((2, BN, F), lambda i: (0, i, 0)),
        pl.BlockSpec((2, BN, FH), lambda i: (0, i, 0)),
    ]
    args = [h, wemb, bemb.reshape(1, F)]
    for nm in ('A', 'B', 'D', 'E'):
        W, b = p[nm]
        args += [W, b.reshape(1, F)]
    in_specs = [b2(0), w_spec, bias_spec] + [w_spec, bias_spec] * 4
    h1, ah, db, ehp = pl.pallas_call(
        functools.partial(_node_body, emb),
        grid=grid, in_specs=in_specs, out_specs=out_specs, out_shape=outs,
    )(*args)
    if not emb:
        h1 = h
    return h1, ah, db.reshape(2 * N, F), ehp.reshape(2 * N, FH)


def _rank1_body(w_ref, wemb_ref, bemb_ref, wc_ref, bc_ref, e_ref, ce_ref):
    w = w_ref[0, 0, :]
    we = wemb_ref[...]
    be = bemb_ref[...]
    e_full = w[:, None] * we + be
    u = jnp.dot(we, wc_ref[...], preferred_element_type=jnp.float32)
    v = jnp.dot(be, wc_ref[...], preferred_element_type=jnp.float32) + bc_ref[...]
    ce_full = w[:, None] * u + v
    e_ref[0] = e_full[:, :FH]
    e_ref[1] = e_full[:, FH:]
    ce_ref[0] = ce_full[:, :FH]
    ce_ref[1] = ce_full[:, FH:]


def _tc_edge_rank1(edge_weight, emb_e, c_wb):
    E = edge_weight.shape[0]
    BE = 2000
    nb = E // BE
    w3 = edge_weight.reshape(nb, 1, BE)
    We, be = emb_e
    Wc, bc = c_wb
    out3 = pl.BlockSpec((2, BE, FH), lambda i: (0, i, 0))
    e, ce = pl.pallas_call(
        _rank1_body,
        grid=(nb,),
        in_specs=[
            pl.BlockSpec((1, 1, BE), lambda i: (i, 0, 0)),
            pl.BlockSpec((1, F), lambda i: (0, 0)),
            pl.BlockSpec((1, F), lambda i: (0, 0)),
            pl.BlockSpec((F, F), lambda i: (0, 0)),
            pl.BlockSpec((1, F), lambda i: (0, 0)),
        ],
        out_specs=[out3, out3],
        out_shape=[jax.ShapeDtypeStruct((2, E, FH), jnp.float32)] * 2,
    )(w3, We.reshape(1, F), be.reshape(1, F), Wc, bc.reshape(1, F))
    return e.reshape(2 * E, FH), ce.reshape(2 * E, FH)


def _ce_body(e_ref, wc_ref, bc_ref, ce_ref):
    e0 = e_ref[0]
    e1 = e_ref[1]
    ce_full = jnp.dot(e0, wc_ref[:FH, :], preferred_element_type=jnp.float32) \
        + jnp.dot(e1, wc_ref[FH:, :], preferred_element_type=jnp.float32) \
        + bc_ref[...]
    ce_ref[0] = ce_full[:, :FH]
    ce_ref[1] = ce_full[:, FH:]


def _tc_edge_ce(eR, c_wb, E):
    BE = 2000
    nb = E // BE
    e3 = eR.reshape(2, E, FH)
    Wc, bc = c_wb
    b3 = pl.BlockSpec((2, BE, FH), lambda i: (0, i, 0))
    ce = pl.pallas_call(
        _ce_body,
        grid=(nb,),
        in_specs=[b3,
                  pl.BlockSpec((F, F), lambda i: (0, 0)),
                  pl.BlockSpec((1, F), lambda i: (0, 0))],
        out_specs=b3,
        out_shape=jax.ShapeDtypeStruct((2, E, FH), jnp.float32),
    )(e3, Wc, bc.reshape(1, F))
    return ce.reshape(2 * E, FH)


def _combine_body(mlp, h_ref, ah_ref, nd_ref, wm_ref, bm_ref, out_ref):
    num = jnp.concatenate([nd_ref[0, :, :FH], nd_ref[1, :, :FH]], axis=1)
    den = jnp.concatenate([nd_ref[0, :, FH:], nd_ref[1, :, FH:]], axis=1)
    hn = h_ref[...] + jnp.maximum(ah_ref[...] + num / (den + 1e-6), 0.0)
    if mlp:
        out_ref[...] = jnp.dot(hn, wm_ref[...],
                               preferred_element_type=jnp.float32) + bm_ref[...]
    else:
        out_ref[...] = hn


def _tc_combine(h, ah, ndR, mlp_wb=None):
    N = h.shape[0]
    BN = 1000
    nd3 = ndR.reshape(2, N, F)
    mlp = mlp_wb is not None
    if mlp:
        Wm, bm = mlp_wb
        ncls = Wm.shape[1]
        Wmp = jnp.pad(Wm, ((0, 0), (0, F - ncls)))
        bmp = jnp.pad(bm, (0, F - ncls))
    else:
        Wmp = jnp.zeros((F, F), jnp.float32)
        bmp = jnp.zeros((F,), jnp.float32)
    b2 = pl.BlockSpec((BN, F), lambda i: (i, 0))
    out = pl.pallas_call(
        functools.partial(_combine_body, mlp),
        grid=(N // BN,),
        in_specs=[b2, b2,
                  pl.BlockSpec((2, BN, F), lambda i: (0, i, 0)),
                  pl.BlockSpec((F, F), lambda i: (0, 0)),
                  pl.BlockSpec((1, F), lambda i: (0, 0))],
        out_specs=b2,
        out_shape=jax.ShapeDtypeStruct((N, F), jnp.float32),
    )(h, ah, nd3, Wmp, bmp.reshape(1, F))
    if mlp:
        return out[:, :ncls]
    return out


def _sc_layer(db, ehp, ce, ein, srcA, dstA, zeros, N, E):
    epert = E // NS
    nchunks = epert // CHUNK
    rows = (N // NS) // 8 * 8
    tail = N - rows * NS

    mesh = plsc.VectorSubcoreMesh(core_axis_name="c", subcore_axis_name="s",
                                  num_cores=NC, num_subcores=NS)

    @functools.partial(
        pl.kernel,
        out_type=[jax.ShapeDtypeStruct((2 * E, FH), jnp.float32),
                  jax.ShapeDtypeStruct((2 * N, F), jnp.float32)],
        mesh=mesh,
        scratch_types=[
            pltpu.VMEM_SHARED((N, F), jnp.float32),
            pltpu.VMEM((CHUNK,), jnp.int32),
            pltpu.VMEM((CHUNK,), jnp.int32),
            pltpu.VMEM((CHUNK,), jnp.int32),
            pltpu.VMEM((CHUNK, F), jnp.float32),
            pltpu.VMEM((CHUNK, FH), jnp.float32),
            pltpu.VMEM((CHUNK, FH), jnp.float32),
            pltpu.VMEM((CHUNK, FH), jnp.float32),
            pltpu.VMEM((CHUNK, F), jnp.float32),
            pltpu.VMEM((CHUNK, FH), jnp.float32),
            pltpu.SemaphoreType.DMA,
            pltpu.SemaphoreType.DMA,
        ],
        compiler_params=pltpu.CompilerParams(use_tc_tiling_on_sc=False),
    )
    def k(db_h, ehp_h, ce_h, ein_h, src_h, dst_h, z_h, eout_h, nd_h,
          acc, srcv, dstv, dstv2, dbg, eg, cev, einv, msv, eov, sem1, sem2):
        c = lax.axis_index("c")
        s = lax.axis_index("s")
        cN = c * N
        cE = c * E

        pltpu.sync_copy(z_h.at[pl.ds(s * rows, rows)],
                        acc.at[pl.ds(s * rows, rows)])
        if tail:
            @pl.when(s == 0)
            def _():
                pltpu.sync_copy(z_h.at[pl.ds(NS * rows, tail)],
                                acc.at[pl.ds(NS * rows, tail)])
        plsc.subcore_barrier()

        def chunk_body(kk, _):
            base = s * epert + kk * CHUNK
            pltpu.sync_copy(src_h.at[pl.ds(base, CHUNK)], srcv)
            pltpu.sync_copy(dst_h.at[pl.ds(base, CHUNK)], dstv)
            for i in range(CHUNK // 16):
                sl = pl.ds(i * 16, 16)
                srcv[sl] = srcv[sl] + cN
                dstv2[sl] = dstv[sl] + cN
            g1 = pltpu.async_copy(db_h.at[srcv], dbg, sem1)
            g2 = pltpu.async_copy(ehp_h.at[dstv2], eg, sem2)
            pltpu.sync_copy(ce_h.at[pl.ds(cE + base, CHUNK)], cev)
            pltpu.sync_copy(ein_h.at[pl.ds(cE + base, CHUNK)], einv)
            g1.wait()
            g2.wait()

            def row_body(r, _):
                for g in range(FH // 16):
                    sl = pl.ds(g * 16, 16)
                    sh = pl.ds(FH + g * 16, 16)
                    en = cev[r, sl] + dbg[r, sl] + eg[r, sl]
                    sg = 1.0 / (1.0 + jnp.exp(-en))
                    msv[r, sl] = sg * dbg[r, sh]
                    msv[r, sh] = sg
                    eov[r, sl] = einv[r, sl] + jnp.maximum(en, 0.0)
                return 0

            lax.fori_loop(0, CHUNK, row_body, 0, unroll=False)
            pltpu.sync_copy(eov, eout_h.at[pl.ds(cE + base, CHUNK)])
            pltpu.sync_copy(msv, acc.at[dstv], add=True)
            return 0

        lax.fori_loop(0, nchunks, chunk_body, 0, unroll=False)
        plsc.subcore_barrier()
        pltpu.sync_copy(acc.at[pl.ds(s * rows, rows)],
                        nd_h.at[pl.ds(cN + s * rows, rows)])
        if tail:
            @pl.when(s == 0)
            def _():
                pltpu.sync_copy(acc.at[pl.ds(NS * rows, tail)],
                                nd_h.at[pl.ds(cN + NS * rows, tail)])

    return k(db, ehp, ce, ein, srcA, dstA, zeros)


def kernel(h, edge_index, edge_weight, params):
    N = h.shape[0]
    E = edge_index.shape[1]
    zeros = jnp.zeros((N, F), jnp.float32)
    layers = params['layers']

    h, ah, db, ehp = _tc_node(h, layers[0], emb_wb=params['emb_h'])
    einR, ceR = _tc_edge_rank1(edge_weight, params['emb_e'], layers[0]['C'])
    srcA = edge_index[0]
    dstA = edge_index[1]
    eoutR, ndR = _sc_layer(db, ehp, ceR, einR, srcA, dstA, zeros, N, E)
    h = _tc_combine(h, ah, ndR)

    for li in range(1, len(layers)):
        p = layers[li]
        _, ah, db, ehp = _tc_node(h, p)
        ceR = _tc_edge_ce(eoutR, p['C'], E)
        eoutR, ndR = _sc_layer(db, ehp, ceR, eoutR, srcA, dstA, zeros, N, E)
        last = li == len(layers) - 1
        h = _tc_combine(h, ah, ndR, mlp_wb=params['mlp'] if last else None)
    return h


# speedup vs baseline: 1.4553x; 1.2428x over previous
import functools
import jax
import jax.numpy as jnp
from jax import lax
from jax.experimental import pallas as pl
from jax.experimental.pallas import tpu as pltpu
from jax.experimental.pallas import tpu_sc as plsc

NC = 2
NS = 16
F = 128
FH = 64
CHUNK = 80


def _node_body(emb, h_ref, wemb_ref, bemb_ref, wa_ref, ba_ref, wb_ref, bb_ref,
               wd_ref, bd_ref, we_ref, be_ref, h1_ref, ah_ref, db_ref, ehp_ref):
    x = h_ref[...]
    if emb:
        x = jnp.dot(x, wemb_ref[...], preferred_element_type=jnp.float32) \
            + bemb_ref[...]
        h1_ref[...] = x
    ah_ref[...] = jnp.dot(x, wa_ref[...], preferred_element_type=jnp.float32) \
        + ba_ref[...]
    bh = jnp.dot(x, wb_ref[...], preferred_element_type=jnp.float32) + bb_ref[...]
    dh = jnp.dot(x, wd_ref[...], preferred_element_type=jnp.float32) + bd_ref[...]
    eh = jnp.dot(x, we_ref[...], preferred_element_type=jnp.float32) + be_ref[...]
    db_ref[0] = jnp.concatenate([dh[:, :FH], bh[:, :FH]], axis=1)
    db_ref[1] = jnp.concatenate([dh[:, FH:], bh[:, FH:]], axis=1)
    ehp_ref[0] = eh[:, :FH]
    ehp_ref[1] = eh[:, FH:]


def _tc_node(h, p, emb_wb=None):
    N = h.shape[0]
    BN = 1000
    grid = (N // BN,)
    b2 = lambda r: pl.BlockSpec((BN, F), lambda i: (i, 0))
    w_spec = pl.BlockSpec((F, F), lambda i: (0, 0))
    bias_spec = pl.BlockSpec((1, F), lambda i: (0, 0))
    emb = emb_wb is not None
    if emb:
        wemb, bemb = emb_wb
    else:
        wemb = jnp.zeros((F, F), jnp.float32)
        bemb = jnp.zeros((F,), jnp.float32)
    outs = [
        jax.ShapeDtypeStruct((N, F), jnp.float32),
        jax.ShapeDtypeStruct((N, F), jnp.float32),
        jax.ShapeDtypeStruct((2, N, F), jnp.float32),
        jax.ShapeDtypeStruct((2, N, FH), jnp.float32),
    ]
    out_specs = [
        b2(0),
        b2(0),
        pl.BlockSpec((2, BN, F), lambda i: (0, i, 0)),
        pl.BlockSpec((2, BN, FH), lambda i: (0, i, 0)),
    ]
    args = [h, wemb, bemb.reshape(1, F)]
    for nm in ('A', 'B', 'D', 'E'):
        W, b = p[nm]
        args += [W, b.reshape(1, F)]
    in_specs = [b2(0), w_spec, bias_spec] + [w_spec, bias_spec] * 4
    h1, ah, db, ehp = pl.pallas_call(
        functools.partial(_node_body, emb),
        grid=grid, in_specs=in_specs, out_specs=out_specs, out_shape=outs,
    )(*args)
    if not emb:
        h1 = h
    return h1, ah, db.reshape(2 * N, F), ehp.reshape(2 * N, FH)


def _rank1_body(w_ref, wemb_ref, bemb_ref, wc_ref, bc_ref, e_ref, ce_ref):
    w = w_ref[0, 0, :]
    we = wemb_ref[...]
    be = bemb_ref[...]
    e_full = w[:, None] * we + be
    u = jnp.dot(we, wc_ref[...], preferred_element_type=jnp.float32)
    v = jnp.dot(be, wc_ref[...], preferred_element_type=jnp.float32) + bc_ref[...]
    ce_full = w[:, None] * u + v
    e_ref[0] = e_full[:, :FH]
    e_ref[1] = e_full[:, FH:]
    ce_ref[0] = ce_full[:, :FH]
    ce_ref[1] = ce_full[:, FH:]


def _tc_edge_rank1(edge_weight, emb_e, c_wb):
    E = edge_weight.shape[0]
    BE = 2000
    nb = E // BE
    w3 = edge_weight.reshape(nb, 1, BE)
    We, be = emb_e
    Wc, bc = c_wb
    out3 = pl.BlockSpec((2, BE, FH), lambda i: (0, i, 0))
    e, ce = pl.pallas_call(
        _rank1_body,
        grid=(nb,),
        in_specs=[
            pl.BlockSpec((1, 1, BE), lambda i: (i, 0, 0)),
            pl.BlockSpec((1, F), lambda i: (0, 0)),
            pl.BlockSpec((1, F), lambda i: (0, 0)),
            pl.BlockSpec((F, F), lambda i: (0, 0)),
            pl.BlockSpec((1, F), lambda i: (0, 0)),
        ],
        out_specs=[out3, out3],
        out_shape=[jax.ShapeDtypeStruct((2, E, FH), jnp.float32)] * 2,
    )(w3, We.reshape(1, F), be.reshape(1, F), Wc, bc.reshape(1, F))
    return e.reshape(2 * E, FH), ce.reshape(2 * E, FH)


def _ce_body(ep_ref, en_ref, wc_ref, bc_ref, e_ref, ce_ref):
    e0 = ep_ref[0] + jnp.maximum(en_ref[0], 0.0)
    e1 = ep_ref[1] + jnp.maximum(en_ref[1], 0.0)
    ce_full = jnp.dot(e0, wc_ref[:FH, :], preferred_element_type=jnp.float32) \
        + jnp.dot(e1, wc_ref[FH:, :], preferred_element_type=jnp.float32) \
        + bc_ref[...]
    e_ref[0] = e0
    e_ref[1] = e1
    ce_ref[0] = ce_full[:, :FH]
    ce_ref[1] = ce_full[:, FH:]


def _tc_edge_ce(epR, enR, c_wb, E):
    BE = 2000
    nb = E // BE
    ep3 = epR.reshape(2, E, FH)
    en3 = enR.reshape(2, E, FH)
    Wc, bc = c_wb
    b3 = pl.BlockSpec((2, BE, FH), lambda i: (0, i, 0))
    e, ce = pl.pallas_call(
        _ce_body,
        grid=(nb,),
        in_specs=[b3, b3,
                  pl.BlockSpec((F, F), lambda i: (0, 0)),
                  pl.BlockSpec((1, F), lambda i: (0, 0))],
        out_specs=[b3, b3],
        out_shape=[jax.ShapeDtypeStruct((2, E, FH), jnp.float32)] * 2,
    )(ep3, en3, Wc, bc.reshape(1, F))
    return e.reshape(2 * E, FH), ce.reshape(2 * E, FH)


def _combine_body(mlp, h_ref, ah_ref, nd_ref, wm_ref, bm_ref, out_ref):
    num = jnp.concatenate([nd_ref[0, :, :FH], nd_ref[1, :, :FH]], axis=1)
    den = jnp.concatenate([nd_ref[0, :, FH:], nd_ref[1, :, FH:]], axis=1)
    hn = h_ref[...] + jnp.maximum(ah_ref[...] + num / (den + 1e-6), 0.0)
    if mlp:
        out_ref[...] = jnp.dot(hn, wm_ref[...],
                               preferred_element_type=jnp.float32) + bm_ref[...]
    else:
        out_ref[...] = hn


def _tc_combine(h, ah, ndR, mlp_wb=None):
    N = h.shape[0]
    BN = 1000
    nd3 = ndR.reshape(2, N, F)
    mlp = mlp_wb is not None
    if mlp:
        Wm, bm = mlp_wb
        ncls = Wm.shape[1]
        Wmp = jnp.pad(Wm, ((0, 0), (0, F - ncls)))
        bmp = jnp.pad(bm, (0, F - ncls))
    else:
        Wmp = jnp.zeros((F, F), jnp.float32)
        bmp = jnp.zeros((F,), jnp.float32)
    b2 = pl.BlockSpec((BN, F), lambda i: (i, 0))
    out = pl.pallas_call(
        functools.partial(_combine_body, mlp),
        grid=(N // BN,),
        in_specs=[b2, b2,
                  pl.BlockSpec((2, BN, F), lambda i: (0, i, 0)),
                  pl.BlockSpec((F, F), lambda i: (0, 0)),
                  pl.BlockSpec((1, F), lambda i: (0, 0))],
        out_specs=b2,
        out_shape=jax.ShapeDtypeStruct((N, F), jnp.float32),
    )(h, ah, nd3, Wmp, bmp.reshape(1, F))
    if mlp:
        return out[:, :ncls]
    return out


def _sc_layer(db, ehp, ce, srcA, dstA, zeros, N, E):
    epert = E // NS
    nchunks = epert // CHUNK
    rows = (N // NS) // 8 * 8
    tail = N - rows * NS
    HC = CHUNK // 2

    mesh = plsc.VectorSubcoreMesh(core_axis_name="c", subcore_axis_name="s",
                                  num_cores=NC, num_subcores=NS)

    assert nchunks % 2 == 0

    @functools.partial(
        pl.kernel,
        out_type=[jax.ShapeDtypeStruct((2 * E, FH), jnp.float32),
                  jax.ShapeDtypeStruct((2 * N, F), jnp.float32)],
        mesh=mesh,
        scratch_types=[
            pltpu.VMEM_SHARED((N, F), jnp.float32),
            [pltpu.VMEM((CHUNK,), jnp.int32)] * 2,
            [pltpu.VMEM((CHUNK,), jnp.int32)] * 2,
            [pltpu.VMEM((CHUNK, F), jnp.float32)] * 2,
            [pltpu.VMEM((CHUNK, FH), jnp.float32)] * 2,
            [pltpu.VMEM((CHUNK, FH), jnp.float32)] * 2,
            pltpu.VMEM((HC, F), jnp.float32),
            pltpu.VMEM((HC, FH), jnp.float32),
            [pltpu.SemaphoreType.DMA] * 2,
            [pltpu.SemaphoreType.DMA] * 2,
        ],
        compiler_params=pltpu.CompilerParams(use_tc_tiling_on_sc=False),
    )
    def k(db_h, ehp_h, ce_h, src_h, dst_h, z_h, en_h, nd_h,
          acc, srcv, dstv, dbg, eg, cev, msv, env, semA, semB):
        c = lax.axis_index("c")
        s = lax.axis_index("s")
        cN = c * N
        cE = c * E

        pltpu.sync_copy(z_h.at[pl.ds(s * rows, rows)],
                        acc.at[pl.ds(s * rows, rows)])
        if tail:
            @pl.when(s == 0)
            def _():
                pltpu.sync_copy(z_h.at[pl.ds(NS * rows, tail)],
                                acc.at[pl.ds(NS * rows, tail)])
        plsc.subcore_barrier()

        def issue_a(kk, b):
            base = s * epert + kk * CHUNK
            pltpu.async_copy(src_h.at[pl.ds(base, CHUNK)], srcv[b], semA[b])
            pltpu.async_copy(dst_h.at[pl.ds(base, CHUNK)], dstv[b], semA[b])
            pltpu.async_copy(ce_h.at[pl.ds(cE + base, CHUNK)], cev[b], semA[b])

        def wait_a(b):
            pltpu.make_async_copy(src_h.at[pl.ds(0, CHUNK)], srcv[b],
                                  semA[b]).wait()
            pltpu.make_async_copy(dst_h.at[pl.ds(0, CHUNK)], dstv[b],
                                  semA[b]).wait()
            pltpu.make_async_copy(ce_h.at[pl.ds(0, CHUNK)], cev[b],
                                  semA[b]).wait()

        def issue_b(b):
            for i in range(CHUNK // 16):
                sl = pl.ds(i * 16, 16)
                srcv[b][sl] = srcv[b][sl] + cN
                dstv[b][sl] = dstv[b][sl] + cN
            pltpu.async_copy(db_h.at[srcv[b]], dbg[b], semB[b])
            pltpu.async_copy(ehp_h.at[dstv[b]], eg[b], semB[b])

        def wait_b(b):
            pltpu.make_async_copy(db_h.at[srcv[b]], dbg[b], semB[b]).wait()
            pltpu.make_async_copy(ehp_h.at[dstv[b]], eg[b], semB[b]).wait()

        def compute(kk, b):
            base = s * epert + kk * CHUNK
            for i in range(CHUNK // 16):
                sl = pl.ds(i * 16, 16)
                dstv[b][sl] = dstv[b][sl] - cN
            for half in range(2):
                r0 = half * HC

                def row_body(r, _):
                    for g in range(FH // 16):
                        sl = pl.ds(g * 16, 16)
                        sh = pl.ds(FH + g * 16, 16)
                        en = cev[b][r0 + r, sl] + dbg[b][r0 + r, sl] \
                            + eg[b][r0 + r, sl]
                        sg = 1.0 / (1.0 + jnp.exp(-en))
                        msv[r, sl] = sg * dbg[b][r0 + r, sh]
                        msv[r, sh] = sg
                        env[r, sl] = en
                    return 0

                lax.fori_loop(0, HC, row_body, 0, unroll=2)
                pltpu.sync_copy(env, en_h.at[pl.ds(cE + base + r0, HC)])
                pltpu.sync_copy(msv, acc.at[dstv[b].at[pl.ds(r0, HC)]],
                                add=True)

        issue_a(0, 0)
        issue_a(1, 1)
        wait_a(0)
        issue_b(0)

        def pair_body(kk, _):
            k0 = 2 * kk
            more = kk < (nchunks // 2 - 1)
            wait_a(1)
            issue_b(1)
            wait_b(0)
            compute(k0, 0)

            @pl.when(more)
            def _():
                issue_a(k0 + 2, 0)

            @pl.when(more)
            def _():
                wait_a(0)
                issue_b(0)
            wait_b(1)
            compute(k0 + 1, 1)

            @pl.when(more)
            def _():
                issue_a(k0 + 3, 1)
            return 0

        lax.fori_loop(0, nchunks // 2, pair_body, 0, unroll=False)
        plsc.subcore_barrier()
        pltpu.sync_copy(acc.at[pl.ds(s * rows, rows)],
                        nd_h.at[pl.ds(cN + s * rows, rows)])
        if tail:
            @pl.when(s == 0)
            def _():
                pltpu.sync_copy(acc.at[pl.ds(NS * rows, tail)],
                                nd_h.at[pl.ds(cN + NS * rows, tail)])

    return k(db, ehp, ce, srcA, dstA, zeros)


def kernel(h, edge_index, edge_weight, params):
    N = h.shape[0]
    E = edge_index.shape[1]
    zeros = jnp.zeros((N, F), jnp.float32)
    layers = params['layers']
    srcA = edge_index[0]
    dstA = edge_index[1]

    h, ah, db, ehp = _tc_node(h, layers[0], emb_wb=params['emb_h'])
    eR, ceR = _tc_edge_rank1(edge_weight, params['emb_e'], layers[0]['C'])
    enR, ndR = _sc_layer(db, ehp, ceR, srcA, dstA, zeros, N, E)
    h = _tc_combine(h, ah, ndR)

    for li in range(1, len(layers)):
        p = layers[li]
        _, ah, db, ehp = _tc_node(h, p)
        eR, ceR = _tc_edge_ce(eR, enR, p['C'], E)
        enR, ndR = _sc_layer(db, ehp, ceR, srcA, dstA, zeros, N, E)
        last = li == len(layers) - 1
        h = _tc_combine(h, ah, ndR, mlp_wb=params['mlp'] if last else None)
    return h


# speedup vs baseline: 2.8718x; 1.9733x over previous
import functools
import jax
import jax.numpy as jnp
from jax import lax
from jax.experimental import pallas as pl
from jax.experimental.pallas import tpu as pltpu
from jax.experimental.pallas import tpu_sc as plsc

NC = 2
NS = 16
F = 128
FH = 64
CHUNK = 80


def _node_body(emb, h_ref, wemb_ref, bemb_ref, wa_ref, ba_ref, wb_ref, bb_ref,
               wd_ref, bd_ref, we_ref, be_ref, h1_ref, ah_ref, db_ref, ehp_ref):
    x = h_ref[...]
    if emb:
        x = jnp.dot(x, wemb_ref[...], preferred_element_type=jnp.float32) \
            + bemb_ref[...]
        h1_ref[...] = x
    ah_ref[...] = jnp.dot(x, wa_ref[...], preferred_element_type=jnp.float32) \
        + ba_ref[...]
    bh = jnp.dot(x, wb_ref[...], preferred_element_type=jnp.float32) + bb_ref[...]
    dh = jnp.dot(x, wd_ref[...], preferred_element_type=jnp.float32) + bd_ref[...]
    eh = jnp.dot(x, we_ref[...], preferred_element_type=jnp.float32) + be_ref[...]
    db_ref[0] = jnp.concatenate([dh[:, :FH], bh[:, :FH]], axis=1)
    db_ref[1] = jnp.concatenate([dh[:, FH:], bh[:, FH:]], axis=1)
    ehp_ref[0] = eh[:, :FH]
    ehp_ref[1] = eh[:, FH:]


def _tc_node(h, p, emb_wb=None):
    N = h.shape[0]
    BN = 1000
    grid = (N // BN,)
    b2 = lambda r: pl.BlockSpec((BN, F), lambda i: (i, 0))
    w_spec = pl.BlockSpec((F, F), lambda i: (0, 0))
    bias_spec = pl.BlockSpec((1, F), lambda i: (0, 0))
    emb = emb_wb is not None
    if emb:
        wemb, bemb = emb_wb
    else:
        wemb = jnp.zeros((F, F), jnp.float32)
        bemb = jnp.zeros((F,), jnp.float32)
    outs = [
        jax.ShapeDtypeStruct((N, F), jnp.float32),
        jax.ShapeDtypeStruct((N, F), jnp.float32),
        jax.ShapeDtypeStruct((2, N, F), jnp.float32),
        jax.ShapeDtypeStruct((2, N, FH), jnp.float32),
    ]
    out_specs = [
        b2(0),
        b2(0),
        pl.BlockSpec((2, BN, F), lambda i: (0, i, 0)),
        pl.BlockSpec((2, BN, FH), lambda i: (0, i, 0)),
    ]
    args = [h, wemb, bemb.reshape(1, F)]
    for nm in ('A', 'B', 'D', 'E'):
        W, b = p[nm]
        args += [W, b.reshape(1, F)]
    in_specs = [b2(0), w_spec, bias_spec] + [w_spec, bias_spec] * 4
    h1, ah, db, ehp = pl.pallas_call(
        functools.partial(_node_body, emb),
        grid=grid, in_specs=in_specs, out_specs=out_specs, out_shape=outs,
    )(*args)
    if not emb:
        h1 = h
    return h1, ah, db.reshape(2 * N, F), ehp.reshape(2 * N, FH)


def _rank1_body(w_ref, wemb_ref, bemb_ref, wc_ref, bc_ref, e_ref, ce_ref):
    w = w_ref[0, 0, :]
    we = wemb_ref[...]
    be = bemb_ref[...]
    e_full = w[:, None] * we + be
    u = jnp.dot(we, wc_ref[...], preferred_element_type=jnp.float32)
    v = jnp.dot(be, wc_ref[...], preferred_element_type=jnp.float32) + bc_ref[...]
    ce_full = w[:, None] * u + v
    e_ref[0] = e_full[:, :FH]
    e_ref[1] = e_full[:, FH:]
    ce_ref[0] = ce_full[:, :FH]
    ce_ref[1] = ce_full[:, FH:]


def _tc_edge_rank1(edge_weight, emb_e, c_wb):
    E = edge_weight.shape[0]
    BE = 2000
    nb = E // BE
    w3 = edge_weight.reshape(nb, 1, BE)
    We, be = emb_e
    Wc, bc = c_wb
    out3 = pl.BlockSpec((2, BE, FH), lambda i: (0, i, 0))
    e, ce = pl.pallas_call(
        _rank1_body,
        grid=(nb,),
        in_specs=[
            pl.BlockSpec((1, 1, BE), lambda i: (i, 0, 0)),
            pl.BlockSpec((1, F), lambda i: (0, 0)),
            pl.BlockSpec((1, F), lambda i: (0, 0)),
            pl.BlockSpec((F, F), lambda i: (0, 0)),
            pl.BlockSpec((1, F), lambda i: (0, 0)),
        ],
        out_specs=[out3, out3],
        out_shape=[jax.ShapeDtypeStruct((2, E, FH), jnp.float32)] * 2,
    )(w3, We.reshape(1, F), be.reshape(1, F), Wc, bc.reshape(1, F))
    return e.reshape(2 * E, FH), ce.reshape(2 * E, FH)


def _ce_body(ep_ref, en_ref, wc_ref, bc_ref, e_ref, ce_ref):
    e0 = ep_ref[0] + jnp.maximum(en_ref[0], 0.0)
    e1 = ep_ref[1] + jnp.maximum(en_ref[1], 0.0)
    ce_full = jnp.dot(e0, wc_ref[:FH, :], preferred_element_type=jnp.float32) \
        + jnp.dot(e1, wc_ref[FH:, :], preferred_element_type=jnp.float32) \
        + bc_ref[...]
    e_ref[0] = e0
    e_ref[1] = e1
    ce_ref[0] = ce_full[:, :FH]
    ce_ref[1] = ce_full[:, FH:]


def _tc_edge_ce(epR, enR, c_wb, E):
    BE = 2000
    nb = E // BE
    ep3 = epR.reshape(2, E, FH)
    en3 = enR.reshape(2, E, FH)
    Wc, bc = c_wb
    b3 = pl.BlockSpec((2, BE, FH), lambda i: (0, i, 0))
    e, ce = pl.pallas_call(
        _ce_body,
        grid=(nb,),
        in_specs=[b3, b3,
                  pl.BlockSpec((F, F), lambda i: (0, 0)),
                  pl.BlockSpec((1, F), lambda i: (0, 0))],
        out_specs=[b3, b3],
        out_shape=[jax.ShapeDtypeStruct((2, E, FH), jnp.float32)] * 2,
    )(ep3, en3, Wc, bc.reshape(1, F))
    return e.reshape(2 * E, FH), ce.reshape(2 * E, FH)


def _combine_body(mlp, h_ref, ah_ref, nd_ref, wm_ref, bm_ref, out_ref):
    num = jnp.concatenate([nd_ref[0, :, :FH], nd_ref[1, :, :FH]], axis=1)
    den = jnp.concatenate([nd_ref[0, :, FH:], nd_ref[1, :, FH:]], axis=1)
    hn = h_ref[...] + jnp.maximum(ah_ref[...] + num / (den + 1e-6), 0.0)
    if mlp:
        out_ref[...] = jnp.dot(hn, wm_ref[...],
                               preferred_element_type=jnp.float32) + bm_ref[...]
    else:
        out_ref[...] = hn


def _tc_combine(h, ah, ndR, mlp_wb=None):
    N = h.shape[0]
    BN = 1000
    nd3 = ndR.reshape(2, N, F)
    mlp = mlp_wb is not None
    if mlp:
        Wm, bm = mlp_wb
        ncls = Wm.shape[1]
        Wmp = jnp.pad(Wm, ((0, 0), (0, F - ncls)))
        bmp = jnp.pad(bm, (0, F - ncls))
    else:
        Wmp = jnp.zeros((F, F), jnp.float32)
        bmp = jnp.zeros((F,), jnp.float32)
    b2 = pl.BlockSpec((BN, F), lambda i: (i, 0))
    out = pl.pallas_call(
        functools.partial(_combine_body, mlp),
        grid=(N // BN,),
        in_specs=[b2, b2,
                  pl.BlockSpec((2, BN, F), lambda i: (0, i, 0)),
                  pl.BlockSpec((F, F), lambda i: (0, 0)),
                  pl.BlockSpec((1, F), lambda i: (0, 0))],
        out_specs=b2,
        out_shape=jax.ShapeDtypeStruct((N, F), jnp.float32),
    )(h, ah, nd3, Wmp, bmp.reshape(1, F))
    if mlp:
        return out[:, :ncls]
    return out


def _sc_layer(db, ehp, ce, srcA, dstA, zeros, N, E):
    epert = E // NS
    nchunks = epert // CHUNK
    rows = (N // NS) // 8 * 8
    tail = N - rows * NS
    HC = CHUNK // 2

    mesh = plsc.VectorSubcoreMesh(core_axis_name="c", subcore_axis_name="s",
                                  num_cores=NC, num_subcores=NS)

    assert nchunks % 2 == 0

    @functools.partial(
        pl.kernel,
        out_type=[jax.ShapeDtypeStruct((2 * E, FH), jnp.float32),
                  jax.ShapeDtypeStruct((2 * N, F), jnp.float32)],
        mesh=mesh,
        scratch_types=[
            pltpu.VMEM_SHARED((N, F), jnp.float32),
            [pltpu.VMEM((CHUNK,), jnp.int32)] * 2,
            [pltpu.VMEM((CHUNK,), jnp.int32)] * 2,
            [pltpu.VMEM((CHUNK, F), jnp.float32)] * 2,
            [pltpu.VMEM((CHUNK, FH), jnp.float32)] * 2,
            [pltpu.VMEM((CHUNK, FH), jnp.float32)] * 2,
            pltpu.VMEM((HC, F), jnp.float32),
            pltpu.VMEM((HC, FH), jnp.float32),
            [pltpu.SemaphoreType.DMA] * 2,
            [pltpu.SemaphoreType.DMA] * 2,
        ],
        compiler_params=pltpu.CompilerParams(use_tc_tiling_on_sc=False),
    )
    def k(db_h, ehp_h, ce_h, src_h, dst_h, z_h, en_h, nd_h,
          acc, srcv, dstv, dbg, eg, cev, msv, env, semA, semB):
        c = lax.axis_index("c")
        s = lax.axis_index("s")
        cN = c * N
        cE = c * E

        pltpu.sync_copy(z_h.at[pl.ds(s * rows, rows)],
                        acc.at[pl.ds(s * rows, rows)])
        if tail:
            @pl.when(s == 0)
            def _():
                pltpu.sync_copy(z_h.at[pl.ds(NS * rows, tail)],
                                acc.at[pl.ds(NS * rows, tail)])
        plsc.subcore_barrier()

        def issue_a(kk, b):
            base = s * epert + kk * CHUNK
            pltpu.async_copy(src_h.at[pl.ds(base, CHUNK)], srcv[b], semA[b])
            pltpu.async_copy(dst_h.at[pl.ds(base, CHUNK)], dstv[b], semA[b])
            pltpu.async_copy(ce_h.at[pl.ds(cE + base, CHUNK)], cev[b], semA[b])

        def wait_a(b):
            pltpu.make_async_copy(src_h.at[pl.ds(0, CHUNK)], srcv[b],
                                  semA[b]).wait()
            pltpu.make_async_copy(dst_h.at[pl.ds(0, CHUNK)], dstv[b],
                                  semA[b]).wait()
            pltpu.make_async_copy(ce_h.at[pl.ds(0, CHUNK)], cev[b],
                                  semA[b]).wait()

        def issue_b(b):
            for i in range(CHUNK // 16):
                sl = pl.ds(i * 16, 16)
                srcv[b][sl] = srcv[b][sl] + cN
                dstv[b][sl] = dstv[b][sl] + cN
            pltpu.async_copy(db_h.at[srcv[b]], dbg[b], semB[b])
            pltpu.async_copy(ehp_h.at[dstv[b]], eg[b], semB[b])

        def wait_b(b):
            pltpu.make_async_copy(db_h.at[srcv[b]], dbg[b], semB[b]).wait()
            pltpu.make_async_copy(ehp_h.at[dstv[b]], eg[b], semB[b]).wait()

        def compute(kk, b):
            base = s * epert + kk * CHUNK
            for i in range(CHUNK // 16):
                sl = pl.ds(i * 16, 16)
                dstv[b][sl] = dstv[b][sl] - cN
            for half in range(2):
                r0 = half * HC

                @plsc.parallel_loop(0, HC, 1, unroll=4)
                def row_body(r):
                    for g in range(FH // 16):
                        sl = pl.ds(g * 16, 16)
                        sh = pl.ds(FH + g * 16, 16)
                        en = cev[b][r0 + r, sl] + dbg[b][r0 + r, sl] \
                            + eg[b][r0 + r, sl]
                        sg = 1.0 / (1.0 + jnp.exp(-en))
                        msv[r, sl] = sg * dbg[b][r0 + r, sh]
                        msv[r, sh] = sg
                        env[r, sl] = en
                pltpu.sync_copy(env, en_h.at[pl.ds(cE + base + r0, HC)])
                pltpu.sync_copy(msv, acc.at[dstv[b].at[pl.ds(r0, HC)]],
                                add=True)

        issue_a(0, 0)
        issue_a(1, 1)
        wait_a(0)
        issue_b(0)

        def pair_body(kk, _):
            k0 = 2 * kk
            more = kk < (nchunks // 2 - 1)
            wait_a(1)
            issue_b(1)
            wait_b(0)
            compute(k0, 0)

            @pl.when(more)
            def _():
                issue_a(k0 + 2, 0)

            @pl.when(more)
            def _():
                wait_a(0)
                issue_b(0)
            wait_b(1)
            compute(k0 + 1, 1)

            @pl.when(more)
            def _():
                issue_a(k0 + 3, 1)
            return 0

        lax.fori_loop(0, nchunks // 2, pair_body, 0, unroll=False)
        plsc.subcore_barrier()
        pltpu.sync_copy(acc.at[pl.ds(s * rows, rows)],
                        nd_h.at[pl.ds(cN + s * rows, rows)])
        if tail:
            @pl.when(s == 0)
            def _():
                pltpu.sync_copy(acc.at[pl.ds(NS * rows, tail)],
                                nd_h.at[pl.ds(cN + NS * rows, tail)])

    return k(db, ehp, ce, srcA, dstA, zeros)


def kernel(h, edge_index, edge_weight, params):
    N = h.shape[0]
    E = edge_index.shape[1]
    zeros = jnp.zeros((N, F), jnp.float32)
    layers = params['layers']
    srcA = edge_index[0]
    dstA = edge_index[1]

    h, ah, db, ehp = _tc_node(h, layers[0], emb_wb=params['emb_h'])
    eR, ceR = _tc_edge_rank1(edge_weight, params['emb_e'], layers[0]['C'])
    enR, ndR = _sc_layer(db, ehp, ceR, srcA, dstA, zeros, N, E)
    h = _tc_combine(h, ah, ndR)

    for li in range(1, len(layers)):
        p = layers[li]
        _, ah, db, ehp = _tc_node(h, p)
        eR, ceR = _tc_edge_ce(eR, enR, p['C'], E)
        enR, ndR = _sc_layer(db, ehp, ceR, srcA, dstA, zeros, N, E)
        last = li == len(layers) - 1
        h = _tc_combine(h, ah, ndR, mlp_wb=params['mlp'] if last else None)
    return h
